# baseline (speedup 1.0000x reference)
import jax
import jax.numpy as jnp
from jax.experimental import pallas as pl
from jax.experimental.pallas import tpu as pltpu

_OUT_LEN = 160000
_LANES = 128
_OUT_TILES = _OUT_LEN // _LANES
_HALF_TILES = _OUT_TILES // 2
_FETCH_TILES = _HALF_TILES + 1


def _crop_kernel(q_ref, m_ref, in_hbm, out_ref, buf, sems):
    b = pl.program_id(0)
    h = pl.program_id(1)
    nb = pl.num_programs(0)
    w = b * 2 + h

    def dma(wi):
        bi, hi = wi // 2, wi % 2
        return pltpu.make_async_copy(
            in_hbm.at[bi, pl.ds(q_ref[bi] + hi * _HALF_TILES, _FETCH_TILES)],
            buf.at[wi].reshape(_FETCH_TILES, 2, _LANES),
            sems.at[wi],
        )

    @pl.when(w == 0)
    def _():
        for wi in range(2 * nb):
            dma(wi).start()

    dma(w).wait()
    x = buf[w]
    m = m_ref[b]
    rot = pltpu.roll(x, (_LANES - m) % _LANES, axis=1)
    lane = jax.lax.broadcasted_iota(jnp.int32, (2 * _HALF_TILES, _LANES), 1)
    res = jnp.where(lane < _LANES - m,
                    rot[0:2 * _HALF_TILES],
                    rot[2:2 * _HALF_TILES + 2])
    out_ref.reshape(2 * _HALF_TILES, _LANES)[...] = res


def kernel(samples):
    B, C, L = samples.shape
    if L < _OUT_LEN:
        return samples
    starts = jax.random.randint(jax.random.key(42), (B,), 0, L - _OUT_LEN)
    q = (starts // _LANES).astype(jnp.int32)
    m = (starts % _LANES).astype(jnp.int32)
    in_view = samples.reshape(B, C, L // _LANES, _LANES).transpose(0, 2, 1, 3)
    out = pl.pallas_call(
        _crop_kernel,
        grid=(B, 2),
        in_specs=[
            pl.BlockSpec(memory_space=pltpu.MemorySpace.SMEM),
            pl.BlockSpec(memory_space=pltpu.MemorySpace.SMEM),
            pl.BlockSpec(memory_space=pl.ANY),
        ],
        out_specs=pl.BlockSpec((1, _HALF_TILES, C, _LANES),
                               lambda b, h: (b, h, 0, 0)),
        out_shape=jax.ShapeDtypeStruct((B, _OUT_TILES, C, _LANES),
                                       samples.dtype),
        scratch_shapes=[
            pltpu.VMEM((2 * B, _FETCH_TILES * 2, _LANES), samples.dtype),
            pltpu.SemaphoreType.DMA((2 * B,)),
        ],
    )(q, m, in_view)
    return out.transpose(0, 2, 1, 3).reshape(B, C, _OUT_LEN)


# speedup vs baseline: 1.3080x; 1.3080x over previous
import jax
import jax.numpy as jnp
from jax.experimental import pallas as pl
from jax.experimental.pallas import tpu as pltpu

_OUT_LEN = 160000
_LANES = 128
_OUT_TILES = _OUT_LEN // _LANES
_FETCH_TILES = _OUT_TILES + 1
_OSLOTS = 4


def _crop_kernel(q_ref, m_ref, in_hbm, out_hbm, buf, obuf, sems, osems):
    g = pl.program_id(0)
    n = pl.num_programs(0)

    def dma(b):
        return pltpu.make_async_copy(
            in_hbm.at[b, pl.ds(q_ref[b], _FETCH_TILES)],
            buf.at[b].reshape(_FETCH_TILES, 2, _LANES),
            sems.at[b],
        )

    def odma(i):
        return pltpu.make_async_copy(
            obuf.at[i % _OSLOTS],
            out_hbm.at[i],
            osems.at[i % _OSLOTS],
        )

    @pl.when(g == 0)
    def _():
        for b in range(n):
            dma(b).start()

    dma(g).wait()
    x = buf[g]
    m = m_ref[g]
    rot = pltpu.roll(x, (_LANES - m) % _LANES, axis=1)
    lane = jax.lax.broadcasted_iota(jnp.int32, (2 * _OUT_TILES, _LANES), 1)
    res = jnp.where(lane < _LANES - m,
                    rot[0:2 * _OUT_TILES],
                    rot[2:2 * _OUT_TILES + 2])

    @pl.when(g >= _OSLOTS)
    def _():
        odma(g - _OSLOTS).wait()

    obuf.at[g % _OSLOTS].reshape(2 * _OUT_TILES, _LANES)[...] = res
    odma(g).start()

    @pl.when(g == n - 1)
    def _():
        for j in range(_OSLOTS - 1, -1, -1):
            odma(g - j).wait()


def kernel(samples):
    B, C, L = samples.shape
    if L < _OUT_LEN:
        return samples
    starts = jax.random.randint(jax.random.key(42), (B,), 0, L - _OUT_LEN)
    q = (starts // _LANES).astype(jnp.int32)
    m = (starts % _LANES).astype(jnp.int32)
    in_view = samples.reshape(B, C, L // _LANES, _LANES).transpose(0, 2, 1, 3)
    out = pl.pallas_call(
        _crop_kernel,
        grid=(B,),
        in_specs=[
            pl.BlockSpec(memory_space=pltpu.MemorySpace.SMEM),
            pl.BlockSpec(memory_space=pltpu.MemorySpace.SMEM),
            pl.BlockSpec(memory_space=pl.ANY),
        ],
        out_specs=pl.BlockSpec(memory_space=pl.ANY),
        out_shape=jax.ShapeDtypeStruct((B, _OUT_TILES, C, _LANES),
                                       samples.dtype),
        scratch_shapes=[
            pltpu.VMEM((B, _FETCH_TILES * 2, _LANES), samples.dtype),
            pltpu.VMEM((_OSLOTS, _OUT_TILES, C, _LANES), samples.dtype),
            pltpu.SemaphoreType.DMA((B,)),
            pltpu.SemaphoreType.DMA((_OSLOTS,)),
        ],
    )(q, m, in_view)
    return out.transpose(0, 2, 1, 3).reshape(B, C, _OUT_LEN)


# speedup vs baseline: 1.3095x; 1.0012x over previous
import jax
import jax.numpy as jnp
from jax.experimental import pallas as pl
from jax.experimental.pallas import tpu as pltpu

_OUT_LEN = 160000
_LANES = 128
_OUT_TILES = _OUT_LEN // _LANES
_FETCH_TILES = _OUT_TILES + 1
_OSLOTS = 12


def _crop_kernel(q_ref, m_ref, in_hbm, out_hbm, buf, obuf, sems, osems):
    g = pl.program_id(0)
    n = pl.num_programs(0)

    def dma(b):
        return pltpu.make_async_copy(
            in_hbm.at[b, pl.ds(q_ref[b], _FETCH_TILES)],
            buf.at[b].reshape(_FETCH_TILES, 2, _LANES),
            sems.at[b],
        )

    def odma(i):
        return pltpu.make_async_copy(
            obuf.at[i % _OSLOTS],
            out_hbm.at[i],
            osems.at[i % _OSLOTS],
        )

    @pl.when(g == 0)
    def _():
        for b in range(n):
            dma(b).start()

    dma(g).wait()
    x = buf[g]
    m = m_ref[g]
    rot = pltpu.roll(x, (_LANES - m) % _LANES, axis=1)
    lane = jax.lax.broadcasted_iota(jnp.int32, (2 * _OUT_TILES, _LANES), 1)
    res = jnp.where(lane < _LANES - m,
                    rot[0:2 * _OUT_TILES],
                    rot[2:2 * _OUT_TILES + 2])

    @pl.when(g >= _OSLOTS)
    def _():
        odma(g - _OSLOTS).wait()

    obuf.at[g % _OSLOTS].reshape(2 * _OUT_TILES, _LANES)[...] = res
    odma(g).start()

    @pl.when(g == n - 1)
    def _():
        for j in range(_OSLOTS - 1, -1, -1):
            odma(g - j).wait()


def kernel(samples):
    B, C, L = samples.shape
    if L < _OUT_LEN:
        return samples
    starts = jax.random.randint(jax.random.key(42), (B,), 0, L - _OUT_LEN)
    q = (starts // _LANES).astype(jnp.int32)
    m = (starts % _LANES).astype(jnp.int32)
    in_view = samples.reshape(B, C, L // _LANES, _LANES).transpose(0, 2, 1, 3)
    out = pl.pallas_call(
        _crop_kernel,
        grid=(B,),
        in_specs=[
            pl.BlockSpec(memory_space=pltpu.MemorySpace.SMEM),
            pl.BlockSpec(memory_space=pltpu.MemorySpace.SMEM),
            pl.BlockSpec(memory_space=pl.ANY),
        ],
        out_specs=pl.BlockSpec(memory_space=pl.ANY),
        out_shape=jax.ShapeDtypeStruct((B, _OUT_TILES, C, _LANES),
                                       samples.dtype),
        scratch_shapes=[
            pltpu.VMEM((B, _FETCH_TILES * 2, _LANES), samples.dtype),
            pltpu.VMEM((_OSLOTS, _OUT_TILES, C, _LANES), samples.dtype),
            pltpu.SemaphoreType.DMA((B,)),
            pltpu.SemaphoreType.DMA((_OSLOTS,)),
        ],
    )(q, m, in_view)
    return out.transpose(0, 2, 1, 3).reshape(B, C, _OUT_LEN)


# speedup vs baseline: 1.3169x; 1.0056x over previous
import jax
import jax.numpy as jnp
from jax.experimental import pallas as pl
from jax.experimental.pallas import tpu as pltpu

_OUT_LEN = 160000
_LANES = 128
_OUT_TILES = _OUT_LEN // _LANES
_FETCH_TILES = _OUT_TILES + 1
_OSLOTS = 8
_HALF = _OUT_TILES // 2


def _crop_kernel(q_ref, m_ref, in_hbm, out_hbm, buf, obuf, sems, osems):
    g = pl.program_id(0)
    n = pl.num_programs(0)

    def dma(b):
        return pltpu.make_async_copy(
            in_hbm.at[b, pl.ds(q_ref[b], _FETCH_TILES)],
            buf.at[b].reshape(_FETCH_TILES, 2, _LANES),
            sems.at[b],
        )

    def odma(i):
        return pltpu.make_async_copy(
            obuf.at[i % _OSLOTS],
            out_hbm.at[i // 2, pl.ds((i % 2) * _HALF, _HALF)],
            osems.at[i % _OSLOTS],
        )

    @pl.when(g == 0)
    def _():
        for b in range(n):
            dma(b).start()

    dma(g).wait()
    x = buf[g]
    m = m_ref[g]
    shift = (_LANES - m) % _LANES
    lane = jax.lax.broadcasted_iota(jnp.int32, (2 * _HALF, _LANES), 1)
    for h in (0, 1):
        xh = x[2 * _HALF * h:2 * _HALF * h + 2 * _HALF + 2]
        rot = pltpu.roll(xh, shift, axis=1)
        res = jnp.where(lane < _LANES - m,
                        rot[0:2 * _HALF],
                        rot[2:2 * _HALF + 2])
        i = g * 2 + h

        @pl.when(i >= _OSLOTS)
        def _():
            odma(i - _OSLOTS).wait()

        obuf.at[i % _OSLOTS].reshape(2 * _HALF, _LANES)[...] = res
        odma(i).start()

    @pl.when(g == n - 1)
    def _():
        for j in range(_OSLOTS - 1, -1, -1):
            odma(2 * n - 1 - j).wait()


def kernel(samples):
    B, C, L = samples.shape
    if L < _OUT_LEN:
        return samples
    starts = jax.random.randint(jax.random.key(42), (B,), 0, L - _OUT_LEN)
    q = (starts // _LANES).astype(jnp.int32)
    m = (starts % _LANES).astype(jnp.int32)
    in_view = samples.reshape(B, C, L // _LANES, _LANES).transpose(0, 2, 1, 3)
    out = pl.pallas_call(
        _crop_kernel,
        grid=(B,),
        in_specs=[
            pl.BlockSpec(memory_space=pltpu.MemorySpace.SMEM),
            pl.BlockSpec(memory_space=pltpu.MemorySpace.SMEM),
            pl.BlockSpec(memory_space=pl.ANY),
        ],
        out_specs=pl.BlockSpec(memory_space=pl.ANY),
        out_shape=jax.ShapeDtypeStruct((B, _OUT_TILES, C, _LANES),
                                       samples.dtype),
        scratch_shapes=[
            pltpu.VMEM((B, _FETCH_TILES * 2, _LANES), samples.dtype),
            pltpu.VMEM((_OSLOTS, _HALF, C, _LANES), samples.dtype),
            pltpu.SemaphoreType.DMA((B,)),
            pltpu.SemaphoreType.DMA((_OSLOTS,)),
        ],
    )(q, m, in_view)
    return out.transpose(0, 2, 1, 3).reshape(B, C, _OUT_LEN)
